# baseline (speedup 1.0000x reference)
import functools

import jax
import jax.numpy as jnp
from jax import lax
from jax.experimental import pallas as pl
from jax.experimental.pallas import tpu as pltpu
from jax.experimental.pallas import tpu_sc as plsc

N = 7680
E = 30720
D = 128
DE = 16
H = 128
TD = 64
MS = 15
NP = 256
NLAYERS = 3
TEMP = 0.1
SINK_ITERS = 20

E2 = 2 * E
NODE_BLK = 512
EDGE_BLK = 2048
GW = 128
CW = 128
PB = 128

_SC_CORES = 2
_SC_SUBCORES = 16
_SC_WORKERS = _SC_CORES * _SC_SUBCORES


def _enc_body(nf, wenc, benc, h_out):
    h_out[...] = jnp.dot(nf[...], wenc[...], preferred_element_type=jnp.float32) + benc[...]


def _encode(nf, wenc, benc):
    grid = (N // NODE_BLK,)
    return pl.pallas_call(
        _enc_body,
        grid=grid,
        in_specs=[
            pl.BlockSpec((NODE_BLK, D), lambda i: (i, 0)),
            pl.BlockSpec((D, D), lambda i: (0, 0)),
            pl.BlockSpec((1, D), lambda i: (0, 0)),
        ],
        out_specs=pl.BlockSpec((NODE_BLK, D), lambda i: (i, 0)),
        out_shape=jax.ShapeDtypeStruct((N, D), jnp.float32),
    )(nf, wenc, benc)


def _msg_body(g, ef, wcomb, bcomb, wft, w2d, b2d, m_out):
    ec = jnp.dot(ef[...], wcomb[...], preferred_element_type=jnp.float32) + bcomb[...]
    gg = jnp.concatenate([g[0], g[1]], axis=1)
    u = jnp.dot(gg, wft[...], preferred_element_type=jnp.float32) + ec
    x = jnp.maximum(u, 0.0)
    m = jnp.dot(x, w2d[...], preferred_element_type=jnp.float32) + b2d[...]
    m_out[0] = m[:, :H]
    m_out[1] = m[:, H:]


def _messages(g3, ef, wcomb, bcomb, wft, w2d, b2d):
    ne = g3.shape[1]
    grid = (ne // EDGE_BLK,)
    return pl.pallas_call(
        _msg_body,
        grid=grid,
        in_specs=[
            pl.BlockSpec((2, EDGE_BLK, D), lambda i: (0, i, 0)),
            pl.BlockSpec((EDGE_BLK, DE), lambda i: (i, 0)),
            pl.BlockSpec((DE, 2 * H), lambda i: (0, 0)),
            pl.BlockSpec((1, 2 * H), lambda i: (0, 0)),
            pl.BlockSpec((2 * D, 2 * H), lambda i: (0, 0)),
            pl.BlockSpec((2 * H, 2 * H), lambda i: (0, 0)),
            pl.BlockSpec((1, 2 * H), lambda i: (0, 0)),
        ],
        out_specs=pl.BlockSpec((2, EDGE_BLK, H), lambda i: (0, i, 0)),
        out_shape=jax.ShapeDtypeStruct((2, ne, H), jnp.float32),
    )(g3, ef, wcomb, bcomb, wft, w2d, b2d)


def _update_final_body(h, pa, pb, nwa, nwb, nb, h_out):
    agg = (pa[0] + pa[1]) + (pb[0] + pb[1])
    h_out[...] = (jnp.dot(h[...], nwa[...], preferred_element_type=jnp.float32)
                  + jnp.dot(agg, nwb[...], preferred_element_type=jnp.float32)
                  + nb[...])


def _update_final(h, parts_a, parts_b, nwa, nwb, nb):
    grid = (N // NODE_BLK,)
    return pl.pallas_call(
        _update_final_body,
        grid=grid,
        in_specs=[
            pl.BlockSpec((NODE_BLK, D), lambda i: (i, 0)),
            pl.BlockSpec((2, NODE_BLK, H), lambda i: (0, i, 0)),
            pl.BlockSpec((2, NODE_BLK, H), lambda i: (0, i, 0)),
            pl.BlockSpec((D, D), lambda i: (0, 0)),
            pl.BlockSpec((H, D), lambda i: (0, 0)),
            pl.BlockSpec((1, D), lambda i: (0, 0)),
        ],
        out_specs=pl.BlockSpec((NODE_BLK, D), lambda i: (i, 0)),
        out_shape=jax.ShapeDtypeStruct((N, D), jnp.float32),
    )(h, parts_a, parts_b, nwa, nwb, nb)


def _sinkhorn_body(x, w1, b1, w2, b2, out, s_ref, pc_ref, hq_ref):
    xb = x[...]
    tt = jnp.maximum(jnp.dot(xb, w1[...], preferred_element_type=jnp.float32) + b1[...], 0.0)
    tb = jnp.dot(tt, w2[...], preferred_element_type=jnp.float32) + b2[...]
    for b in range(PB):
        tq_b = tb[30 * b:30 * b + MS, :]
        tc_b = tb[30 * b + MS:30 * b + 30, :]
        s_ref[b] = lax.dot_general(tq_b, tc_b, (((1,), (1,)), ((), ())),
                                   preferred_element_type=jnp.float32)
        hq_ref[b] = xb[30 * b:30 * b + MS, :]
    s_pm = s_ref[...]

    eye_pb = jnp.eye(PB, dtype=jnp.float32)
    eye_ms = jnp.eye(MS, dtype=jnp.float32)
    la3 = jnp.stack(
        [lax.dot_general(s_pm[:, i, :], eye_pb, (((0,), (0,)), ((), ())),
                         preferred_element_type=jnp.float32)
         for i in range(MS)], axis=0) * (1.0 / TEMP)

    def _iter(_, la):
        m2 = jnp.max(la, axis=1, keepdims=True)
        la = la - (m2 + jnp.log(jnp.sum(jnp.exp(la - m2), axis=1, keepdims=True)))
        m1 = jnp.max(la, axis=0, keepdims=True)
        la = la - (m1 + jnp.log(jnp.sum(jnp.exp(la - m1), axis=0, keepdims=True)))
        return la

    la3 = lax.fori_loop(0, SINK_ITERS, _iter, la3)
    plan3 = jnp.exp(la3)
    plan_pm = jnp.stack(
        [lax.dot_general(plan3[i], eye_ms, (((0,), (0,)), ((), ())),
                         preferred_element_type=jnp.float32)
         for i in range(MS)], axis=1)
    for b in range(PB):
        hc_b = xb[30 * b + MS:30 * b + 30, :]
        pc_ref[b] = jnp.dot(plan_pm[b], hc_b, preferred_element_type=jnp.float32)
    diff = jnp.maximum(hq_ref[...] - pc_ref[...], 0.0)
    r = jnp.sum(jnp.sum(diff, axis=2), axis=1)
    out[...] = (-r).reshape(1, 1, PB)


def _sinkhorn_scores(h, w1, b1, w2, b2):
    grid = (NP // PB,)
    return pl.pallas_call(
        _sinkhorn_body,
        grid=grid,
        in_specs=[
            pl.BlockSpec((PB * 30, D), lambda i: (i, 0)),
            pl.BlockSpec((D, TD), lambda i: (0, 0)),
            pl.BlockSpec((1, TD), lambda i: (0, 0)),
            pl.BlockSpec((TD, TD), lambda i: (0, 0)),
            pl.BlockSpec((1, TD), lambda i: (0, 0)),
        ],
        out_specs=pl.BlockSpec((1, 1, PB), lambda i: (i, 0, 0)),
        out_shape=jax.ShapeDtypeStruct((NP // PB, 1, PB), jnp.float32),
        scratch_shapes=[
            pltpu.VMEM((PB, MS, MS), jnp.float32),
            pltpu.VMEM((PB, MS, D), jnp.float32),
            pltpu.VMEM((PB, MS, D), jnp.float32),
        ],
    )(h, w1, b1, w2, b2)


def _sc_gather(table, idx2d):
    ni = idx2d.shape[1]
    cols = table.shape[1]
    mesh = plsc.VectorSubcoreMesh(core_axis_name="c", subcore_axis_name="s")

    @functools.partial(
        pl.kernel,
        out_type=jax.ShapeDtypeStruct((ni, cols), table.dtype),
        mesh=mesh,
    )
    def k(tab_hbm, i_hbm, o_hbm):
        def body(i_vmem, o_vmem):
            pltpu.sync_copy(tab_hbm.at[i_vmem.at[0]], o_vmem)

        pltpu.emit_pipeline(
            body,
            grid=(ni // GW,),
            in_specs=[pl.BlockSpec((1, GW), lambda i: (0, i))],
            out_specs=[pl.BlockSpec((GW, cols), lambda i: (i, 0))],
            core_axis_name=("c", "s"),
            dimension_semantics=(pltpu.PARALLEL,),
        )(i_hbm, o_hbm)

    return k(table, idx2d)


def _sc_scatter_add(m2, idx2d, zeros_nd):
    nch = idx2d.shape[0] * idx2d.shape[1]
    ch_per_core = nch // _SC_CORES
    ch_per_worker = nch // _SC_WORKERS
    rows_per_sub = N // _SC_SUBCORES
    gch = 1
    ngroup = ch_per_worker // gch
    mesh = plsc.VectorSubcoreMesh(core_axis_name="c", subcore_axis_name="s")

    @functools.partial(
        pl.kernel,
        out_type=jax.ShapeDtypeStruct((_SC_CORES, N, D), jnp.float32),
        mesh=mesh,
        scratch_types=[
            pltpu.VMEM_SHARED((N, D), jnp.float32),
            pltpu.VMEM((ch_per_worker, CW), jnp.int32),
            pltpu.VMEM((gch * CW, D), jnp.float32),
            pltpu.VMEM((gch * CW, D), jnp.float32),
            pltpu.SemaphoreType.DMA,
            pltpu.SemaphoreType.DMA,
        ],
    )
    def k(m_hbm, i_hbm, z_hbm, o_hbm, acc_shared, idx_v, mb0, mb1, sem0, sem1):
        c = lax.axis_index("c")
        s = lax.axis_index("s")
        row0 = s * rows_per_sub
        wid = c * _SC_SUBCORES + s
        base_chunk = c * ch_per_core + s * ch_per_worker
        cp_init = pltpu.async_copy(z_hbm.at[pl.ds(row0, rows_per_sub)],
                                   acc_shared.at[pl.ds(row0, rows_per_sub)], sem1)
        pltpu.sync_copy(i_hbm.at[wid], idx_v)
        bufs = (mb0, mb1)
        sems = (sem0, sem1)
        cp = pltpu.async_copy(m_hbm.at[pl.ds(base_chunk * CW, gch * CW)], mb0, sem0)
        cp_init.wait()
        plsc.subcore_barrier()
        for g in range(ngroup):
            cp.wait()
            if g + 1 < ngroup:
                nxt = (base_chunk + (g + 1) * gch) * CW
                cp = pltpu.async_copy(m_hbm.at[pl.ds(nxt, gch * CW)],
                                      bufs[(g + 1) % 2], sems[(g + 1) % 2])
            buf = bufs[g % 2]
            for j in range(gch):
                pltpu.sync_copy(buf.at[pl.ds(j * CW, CW)],
                                acc_shared.at[idx_v.at[g * gch + j]], add=True)
        plsc.subcore_barrier()
        pltpu.sync_copy(acc_shared.at[pl.ds(row0, rows_per_sub)],
                        o_hbm.at[c, pl.ds(row0, rows_per_sub)])

    return k(m2, idx2d, zeros_nd)


def kernel(node_features, edge_features, from_idx, to_idx,
           enc_node_W, enc_node_b, enc_edge_W, enc_edge_b,
           msg_W1, msg_b1, msg_W2, msg_b2,
           rmsg_W1, rmsg_b1, rmsg_W2, rmsg_b2,
           node_W, node_b, fc1_W, fc1_b, fc2_W, fc2_b):
    f32 = jnp.float32
    from_i = from_idx.astype(jnp.int32)
    to_i = to_idx.astype(jnp.int32)

    wf = jnp.concatenate([msg_W1[:D], rmsg_W1[D:2 * D]], axis=1)
    wt = jnp.concatenate([msg_W1[D:2 * D], rmsg_W1[:D]], axis=1)
    wft = jnp.concatenate([wf, wt], axis=0)
    wcc = jnp.concatenate([msg_W1[2 * D:], rmsg_W1[2 * D:]], axis=1)
    bcc = jnp.concatenate([msg_b1, rmsg_b1]).reshape(1, 2 * H)
    wcomb = enc_edge_W @ wcc
    bcomb = enc_edge_b.reshape(1, DE) @ wcc + bcc
    zh = jnp.zeros((H, H), f32)
    w2d = jnp.concatenate(
        [jnp.concatenate([msg_W2, zh], axis=1),
         jnp.concatenate([zh, rmsg_W2], axis=1)], axis=0)
    b2d = jnp.concatenate([msg_b2, rmsg_b2]).reshape(1, 2 * H)
    nwa = node_W[:D]
    nwb = node_W[D:]
    benc = enc_node_b.reshape(1, D)
    nb = node_b.reshape(1, D)
    fb1 = fc1_b.reshape(1, TD)
    fb2 = fc2_b.reshape(1, TD)

    ea = 16384
    eb = E - ea
    fa, fb = from_i[:ea], from_i[ea:]
    ta, tb = to_i[:ea], to_i[ea:]
    gat_a = jnp.concatenate([fa, ta]).reshape(1, 2 * ea)
    gat_b = jnp.concatenate([fb, tb]).reshape(1, 2 * eb)
    sct_a = jnp.concatenate([ta, fa]).reshape(_SC_WORKERS, (2 * ea) // (_SC_WORKERS * CW), CW)
    sct_b = jnp.concatenate([tb, fb]).reshape(_SC_WORKERS, (2 * eb) // (_SC_WORKERS * CW), CW)
    ef_a = edge_features[:ea]
    ef_b = edge_features[ea:]
    zeros_nd = jnp.zeros((N, D), f32)

    h = _encode(node_features, enc_node_W, benc)

    for layer in range(NLAYERS):
        ga = _sc_gather(h, gat_a)
        gb = _sc_gather(h, gat_b)
        ma = _messages(ga.reshape(2, ea, D), ef_a, wcomb, bcomb, wft, w2d, b2d)
        mb = _messages(gb.reshape(2, eb, D), ef_b, wcomb, bcomb, wft, w2d, b2d)
        pa = _sc_scatter_add(ma.reshape(2 * ea, H), sct_a, zeros_nd)
        pb = _sc_scatter_add(mb.reshape(2 * eb, H), sct_b, zeros_nd)
        h = _update_final(h, pa, pb, nwa, nwb, nb)

    scores = _sinkhorn_scores(h, fc1_W, fb1, fc2_W, fb2)
    return scores.reshape(NP)


# speedup vs baseline: 1.0635x; 1.0635x over previous
import functools

import jax
import jax.numpy as jnp
from jax import lax
from jax.experimental import pallas as pl
from jax.experimental.pallas import tpu as pltpu
from jax.experimental.pallas import tpu_sc as plsc

N = 7680
E = 30720
D = 128
DE = 16
H = 128
TD = 64
MS = 15
NP = 256
NLAYERS = 3
TEMP = 0.1
SINK_ITERS = 20

E2 = 2 * E
NODE_BLK = 512
EDGE_BLK = 2048
GW = 128
CW = 128
PB = 128

_SC_CORES = 2
_SC_SUBCORES = 16
_SC_WORKERS = _SC_CORES * _SC_SUBCORES


def _enc_body(nf, wenc, benc, h_out):
    h_out[...] = jnp.dot(nf[...], wenc[...], preferred_element_type=jnp.float32) + benc[...]


def _encode(nf, wenc, benc):
    grid = (N // NODE_BLK,)
    return pl.pallas_call(
        _enc_body,
        grid=grid,
        in_specs=[
            pl.BlockSpec((NODE_BLK, D), lambda i: (i, 0)),
            pl.BlockSpec((D, D), lambda i: (0, 0)),
            pl.BlockSpec((1, D), lambda i: (0, 0)),
        ],
        out_specs=pl.BlockSpec((NODE_BLK, D), lambda i: (i, 0)),
        out_shape=jax.ShapeDtypeStruct((N, D), jnp.float32),
    )(nf, wenc, benc)


def _msg_body(g, ef, wcomb, bcomb, wft, w2d, b2d, m_out):
    ec = jnp.dot(ef[...], wcomb[...], preferred_element_type=jnp.float32) + bcomb[...]
    gg = jnp.concatenate([g[0], g[1]], axis=1)
    u = jnp.dot(gg, wft[...], preferred_element_type=jnp.float32) + ec
    x = jnp.maximum(u, 0.0)
    m = jnp.dot(x, w2d[...], preferred_element_type=jnp.float32) + b2d[...]
    m_out[0] = m[:, :H]
    m_out[1] = m[:, H:]


def _messages(g3, ef, wcomb, bcomb, wft, w2d, b2d):
    ne = g3.shape[1]
    grid = (ne // EDGE_BLK,)
    return pl.pallas_call(
        _msg_body,
        grid=grid,
        in_specs=[
            pl.BlockSpec((2, EDGE_BLK, D), lambda i: (0, i, 0)),
            pl.BlockSpec((EDGE_BLK, DE), lambda i: (i, 0)),
            pl.BlockSpec((DE, 2 * H), lambda i: (0, 0)),
            pl.BlockSpec((1, 2 * H), lambda i: (0, 0)),
            pl.BlockSpec((2 * D, 2 * H), lambda i: (0, 0)),
            pl.BlockSpec((2 * H, 2 * H), lambda i: (0, 0)),
            pl.BlockSpec((1, 2 * H), lambda i: (0, 0)),
        ],
        out_specs=pl.BlockSpec((2, EDGE_BLK, H), lambda i: (0, i, 0)),
        out_shape=jax.ShapeDtypeStruct((2, ne, H), jnp.float32),
    )(g3, ef, wcomb, bcomb, wft, w2d, b2d)


def _update_final_body(h, p, nwa, nwb, nb, h_out):
    agg = p[0] + p[1]
    h_out[...] = (jnp.dot(h[...], nwa[...], preferred_element_type=jnp.float32)
                  + jnp.dot(agg, nwb[...], preferred_element_type=jnp.float32)
                  + nb[...])


def _update_final(h, parts, nwa, nwb, nb):
    grid = (N // NODE_BLK,)
    return pl.pallas_call(
        _update_final_body,
        grid=grid,
        in_specs=[
            pl.BlockSpec((NODE_BLK, D), lambda i: (i, 0)),
            pl.BlockSpec((2, NODE_BLK, H), lambda i: (0, i, 0)),
            pl.BlockSpec((D, D), lambda i: (0, 0)),
            pl.BlockSpec((H, D), lambda i: (0, 0)),
            pl.BlockSpec((1, D), lambda i: (0, 0)),
        ],
        out_specs=pl.BlockSpec((NODE_BLK, D), lambda i: (i, 0)),
        out_shape=jax.ShapeDtypeStruct((N, D), jnp.float32),
    )(h, parts, nwa, nwb, nb)


def _sinkhorn_body(x, w1, b1, w2, b2, out, s_ref, pc_ref, hq_ref):
    xb = x[...]
    tt = jnp.maximum(jnp.dot(xb, w1[...], preferred_element_type=jnp.float32) + b1[...], 0.0)
    tb = jnp.dot(tt, w2[...], preferred_element_type=jnp.float32) + b2[...]
    for b in range(PB):
        tq_b = tb[30 * b:30 * b + MS, :]
        tc_b = tb[30 * b + MS:30 * b + 30, :]
        s_ref[b] = lax.dot_general(tq_b, tc_b, (((1,), (1,)), ((), ())),
                                   preferred_element_type=jnp.float32)
        hq_ref[b] = xb[30 * b:30 * b + MS, :]
    s_pm = s_ref[...]

    eye_pb = jnp.eye(PB, dtype=jnp.float32)
    eye_ms = jnp.eye(MS, dtype=jnp.float32)
    la3 = jnp.stack(
        [lax.dot_general(s_pm[:, i, :], eye_pb, (((0,), (0,)), ((), ())),
                         preferred_element_type=jnp.float32)
         for i in range(MS)], axis=0) * (1.0 / TEMP)

    def _iter(_, la):
        m2 = jnp.max(la, axis=1, keepdims=True)
        la = la - (m2 + jnp.log(jnp.sum(jnp.exp(la - m2), axis=1, keepdims=True)))
        m1 = jnp.max(la, axis=0, keepdims=True)
        la = la - (m1 + jnp.log(jnp.sum(jnp.exp(la - m1), axis=0, keepdims=True)))
        return la

    la3 = lax.fori_loop(0, SINK_ITERS, _iter, la3)
    plan3 = jnp.exp(la3)
    plan_pm = jnp.stack(
        [lax.dot_general(plan3[i], eye_ms, (((0,), (0,)), ((), ())),
                         preferred_element_type=jnp.float32)
         for i in range(MS)], axis=1)
    for b in range(PB):
        hc_b = xb[30 * b + MS:30 * b + 30, :]
        pc_ref[b] = jnp.dot(plan_pm[b], hc_b, preferred_element_type=jnp.float32)
    diff = jnp.maximum(hq_ref[...] - pc_ref[...], 0.0)
    r = jnp.sum(jnp.sum(diff, axis=2), axis=1)
    out[...] = (-r).reshape(1, 1, PB)


def _sinkhorn_scores(h, w1, b1, w2, b2):
    grid = (NP // PB,)
    return pl.pallas_call(
        _sinkhorn_body,
        grid=grid,
        in_specs=[
            pl.BlockSpec((PB * 30, D), lambda i: (i, 0)),
            pl.BlockSpec((D, TD), lambda i: (0, 0)),
            pl.BlockSpec((1, TD), lambda i: (0, 0)),
            pl.BlockSpec((TD, TD), lambda i: (0, 0)),
            pl.BlockSpec((1, TD), lambda i: (0, 0)),
        ],
        out_specs=pl.BlockSpec((1, 1, PB), lambda i: (i, 0, 0)),
        out_shape=jax.ShapeDtypeStruct((NP // PB, 1, PB), jnp.float32),
        scratch_shapes=[
            pltpu.VMEM((PB, MS, MS), jnp.float32),
            pltpu.VMEM((PB, MS, D), jnp.float32),
            pltpu.VMEM((PB, MS, D), jnp.float32),
        ],
    )(h, w1, b1, w2, b2)


def _sc_gather(table, idx3d):
    nw = idx3d.shape[1]
    ni = _SC_WORKERS * nw * GW
    cols = table.shape[1]
    mesh = plsc.VectorSubcoreMesh(core_axis_name="c", subcore_axis_name="s")

    @functools.partial(
        pl.kernel,
        out_type=jax.ShapeDtypeStruct((ni, cols), table.dtype),
        mesh=mesh,
        scratch_types=[
            pltpu.VMEM((nw, GW), jnp.int32),
            pltpu.VMEM((GW, cols), jnp.float32),
            pltpu.VMEM((GW, cols), jnp.float32),
            pltpu.SemaphoreType.DMA,
            pltpu.SemaphoreType.DMA,
            pltpu.SemaphoreType.DMA,
            pltpu.SemaphoreType.DMA,
        ],
    )
    def k(tab_hbm, i_hbm, o_hbm, idx_v, b0, b1, sg0, sg1, sw0, sw1):
        c = lax.axis_index("c")
        s = lax.axis_index("s")
        wid = c * _SC_SUBCORES + s
        base_row = wid * nw * GW
        pltpu.sync_copy(i_hbm.at[wid], idx_v)
        bufs = (b0, b1)
        sgs = (sg0, sg1)
        sws = (sw0, sw1)
        cpg = [None, None]
        cpw = [None, None]
        cpg[0] = pltpu.async_copy(tab_hbm.at[idx_v.at[0]], b0, sg0)
        for j in range(nw):
            k_ = j % 2
            if j + 1 < nw:
                if j >= 1:
                    cpw[(j + 1) % 2].wait()
                cpg[(j + 1) % 2] = pltpu.async_copy(
                    tab_hbm.at[idx_v.at[j + 1]], bufs[(j + 1) % 2], sgs[(j + 1) % 2])
            cpg[k_].wait()
            cpw[k_] = pltpu.async_copy(
                bufs[k_], o_hbm.at[pl.ds(base_row + j * GW, GW)], sws[k_])
        cpw[(nw - 1) % 2].wait()
        if nw >= 2:
            cpw[(nw - 2) % 2].wait()

    return k(table, idx3d)


def _sc_scatter_add(m2, idx2d, zeros_nd):
    nch = idx2d.shape[0] * idx2d.shape[1]
    ch_per_core = nch // _SC_CORES
    ch_per_worker = nch // _SC_WORKERS
    rows_per_sub = N // _SC_SUBCORES
    gch = 1
    ngroup = ch_per_worker // gch
    mesh = plsc.VectorSubcoreMesh(core_axis_name="c", subcore_axis_name="s")

    @functools.partial(
        pl.kernel,
        out_type=jax.ShapeDtypeStruct((_SC_CORES, N, D), jnp.float32),
        mesh=mesh,
        scratch_types=[
            pltpu.VMEM_SHARED((N, D), jnp.float32),
            pltpu.VMEM((ch_per_worker, CW), jnp.int32),
            pltpu.VMEM((gch * CW, D), jnp.float32),
            pltpu.VMEM((gch * CW, D), jnp.float32),
            pltpu.SemaphoreType.DMA,
            pltpu.SemaphoreType.DMA,
        ],
    )
    def k(m_hbm, i_hbm, z_hbm, o_hbm, acc_shared, idx_v, mb0, mb1, sem0, sem1):
        c = lax.axis_index("c")
        s = lax.axis_index("s")
        row0 = s * rows_per_sub
        wid = c * _SC_SUBCORES + s
        base_chunk = c * ch_per_core + s * ch_per_worker
        cp_init = pltpu.async_copy(z_hbm.at[pl.ds(row0, rows_per_sub)],
                                   acc_shared.at[pl.ds(row0, rows_per_sub)], sem1)
        pltpu.sync_copy(i_hbm.at[wid], idx_v)
        bufs = (mb0, mb1)
        sems = (sem0, sem1)
        cp = pltpu.async_copy(m_hbm.at[pl.ds(base_chunk * CW, gch * CW)], mb0, sem0)
        cp_init.wait()
        plsc.subcore_barrier()
        for g in range(ngroup):
            cp.wait()
            if g + 1 < ngroup:
                nxt = (base_chunk + (g + 1) * gch) * CW
                cp = pltpu.async_copy(m_hbm.at[pl.ds(nxt, gch * CW)],
                                      bufs[(g + 1) % 2], sems[(g + 1) % 2])
            buf = bufs[g % 2]
            for j in range(gch):
                pltpu.sync_copy(buf.at[pl.ds(j * CW, CW)],
                                acc_shared.at[idx_v.at[g * gch + j]], add=True)
        plsc.subcore_barrier()
        pltpu.sync_copy(acc_shared.at[pl.ds(row0, rows_per_sub)],
                        o_hbm.at[c, pl.ds(row0, rows_per_sub)])

    return k(m2, idx2d, zeros_nd)


def kernel(node_features, edge_features, from_idx, to_idx,
           enc_node_W, enc_node_b, enc_edge_W, enc_edge_b,
           msg_W1, msg_b1, msg_W2, msg_b2,
           rmsg_W1, rmsg_b1, rmsg_W2, rmsg_b2,
           node_W, node_b, fc1_W, fc1_b, fc2_W, fc2_b):
    f32 = jnp.float32
    from_i = from_idx.astype(jnp.int32)
    to_i = to_idx.astype(jnp.int32)

    wf = jnp.concatenate([msg_W1[:D], rmsg_W1[D:2 * D]], axis=1)
    wt = jnp.concatenate([msg_W1[D:2 * D], rmsg_W1[:D]], axis=1)
    wft = jnp.concatenate([wf, wt], axis=0)
    wcc = jnp.concatenate([msg_W1[2 * D:], rmsg_W1[2 * D:]], axis=1)
    bcc = jnp.concatenate([msg_b1, rmsg_b1]).reshape(1, 2 * H)
    wcomb = enc_edge_W @ wcc
    bcomb = enc_edge_b.reshape(1, DE) @ wcc + bcc
    zh = jnp.zeros((H, H), f32)
    w2d = jnp.concatenate(
        [jnp.concatenate([msg_W2, zh], axis=1),
         jnp.concatenate([zh, rmsg_W2], axis=1)], axis=0)
    b2d = jnp.concatenate([msg_b2, rmsg_b2]).reshape(1, 2 * H)
    nwa = node_W[:D]
    nwb = node_W[D:]
    benc = enc_node_b.reshape(1, D)
    nb = node_b.reshape(1, D)
    fb1 = fc1_b.reshape(1, TD)
    fb2 = fc2_b.reshape(1, TD)

    gat_idx = jnp.concatenate([from_i, to_i]).reshape(
        _SC_WORKERS, E2 // (_SC_WORKERS * GW), GW)
    sct_idx = jnp.concatenate([to_i, from_i]).reshape(
        _SC_WORKERS, E2 // (_SC_WORKERS * CW), CW)
    zeros_nd = jnp.zeros((N, D), f32)

    h = _encode(node_features, enc_node_W, benc)

    for layer in range(NLAYERS):
        g = _sc_gather(h, gat_idx)
        m = _messages(g.reshape(2, E, D), edge_features, wcomb, bcomb,
                      wft, w2d, b2d)
        parts = _sc_scatter_add(m.reshape(E2, H), sct_idx, zeros_nd)
        h = _update_final(h, parts, nwa, nwb, nb)

    scores = _sinkhorn_scores(h, fc1_W, fb1, fc2_W, fb2)
    return scores.reshape(NP)
